# baseline (speedup 1.0000x reference)
import functools
import math

import jax
import jax.numpy as jnp
from jax import lax
from jax.experimental import pallas as pl
from jax.experimental.pallas import tpu as pltpu
from jax.experimental.pallas import tpu_sc as plsc

D_MODEL = 64
SCALE = math.sqrt(D_MODEL)
NC, NS = 2, 16
NW = NC * NS
LANES = 16
C = 256
NBUF = 2

B0, B1 = 4096, 200
TOTAL = B0 * B1
NCHUNK = TOTAL // C
CH_PER_W = NCHUNK // NW
QPB = B0 // C


@jax.jit
def _emb_lookup(x_t, table):
    mesh = plsc.VectorSubcoreMesh(core_axis_name="c", subcore_axis_name="s")

    @functools.partial(
        pl.kernel,
        out_type=jax.ShapeDtypeStruct((B1, 8, B0 // 128, 8, 128), jnp.float32),
        mesh=mesh,
        scratch_types=(
            [pltpu.VMEM((C,), jnp.int32) for _ in range(NBUF)]
            + [pltpu.VMEM((C, D_MODEL), jnp.float32) for _ in range(NBUF)]
            + [pltpu.VMEM((8, C // 128, 8, 128), jnp.float32) for _ in range(NBUF)]
            + [pltpu.SemaphoreType.DMA for _ in range(2 * NBUF)]
        ),
        compiler_params=pltpu.CompilerParams(use_tc_tiling_on_sc=False, needs_layout_passes=False),
    )
    def body(x_hbm, table_hbm, out_hbm, *scratch):
        idx = scratch[:NBUF]
        rows = scratch[NBUF:2 * NBUF]
        tbuf = scratch[2 * NBUF:3 * NBUF]
        gsem = scratch[3 * NBUF:4 * NBUF]
        osem = scratch[4 * NBUF:]

        wid = lax.axis_index("s") * NC + lax.axis_index("c")
        chunk0 = wid * CH_PER_W
        iota = lax.iota(jnp.int32, LANES)

        def start_gather(b, gc):
            start = pl.multiple_of(gc * C, C)
            pltpu.sync_copy(x_hbm.at[pl.ds(start, C)], idx[b])
            pltpu.async_copy(table_hbm.at[idx[b]], rows[b], gsem[b])

        def wait_gather(b):
            pltpu.make_async_copy(table_hbm.at[idx[b]], rows[b], gsem[b]).wait()

        def transpose_scale(b):
            def jbody(j, carry):
                g2 = j // 8
                j16 = j % 8
                rvec = g2 * 128 + j16 * 16 + iota
                for d in range(D_MODEL):
                    v = plsc.load_gather(
                        rows[b], [rvec, jnp.full((LANES,), d, jnp.int32)]
                    )
                    tbuf[b][d // 8, g2, d % 8, pl.ds(j16 * 16, LANES)] = v * SCALE
                return carry
            lax.fori_loop(0, (C // 128) * 8, jbody, 0)

        def write_out(b, b1, q):
            for db in range(8):
                pltpu.async_copy(
                    tbuf[b].at[db],
                    out_hbm.at[b1, db, pl.ds((C // 128) * q, C // 128)],
                    osem[b],
                )

        def wait_writes(b, b1, q):
            for db in range(8):
                pltpu.make_async_copy(
                    tbuf[b].at[db],
                    out_hbm.at[b1, db, pl.ds((C // 128) * q, C // 128)],
                    osem[b],
                ).wait()

        def process(b, gc, do_wait_writes):
            b1 = gc // QPB
            q = gc % QPB
            wait_gather(b)
            if do_wait_writes:
                wait_writes(b, b1, q)
            transpose_scale(b)
            write_out(b, b1, q)

        for b in range(NBUF):
            start_gather(b, chunk0 + b)
        for b in range(NBUF):
            process(b, chunk0 + b, do_wait_writes=False)
            start_gather(b, chunk0 + b + NBUF)

        def main(i, carry):
            for b in range(NBUF):
                gc = chunk0 + i * NBUF + b
                process(b, gc, do_wait_writes=True)
                start_gather(b, gc + NBUF)
            return carry

        lax.fori_loop(1, CH_PER_W // NBUF - 1, main, 0)

        for b in range(NBUF):
            gc = chunk0 + CH_PER_W - NBUF + b
            process(b, gc, do_wait_writes=True)
        for b in range(NBUF):
            gc = chunk0 + CH_PER_W - NBUF + b
            wait_writes(b, gc // QPB, gc % QPB)

    return body(x_t, table)


def kernel(x, table):
    x_t = x.T.reshape(TOTAL)
    out5 = _emb_lookup(x_t, table)
    return out5.transpose(2, 4, 0, 1, 3).reshape(B0, B1, D_MODEL)


# speedup vs baseline: 1.4447x; 1.4447x over previous
import functools
import math

import jax
import jax.numpy as jnp
from jax import lax
from jax.experimental import pallas as pl
from jax.experimental.pallas import tpu as pltpu
from jax.experimental.pallas import tpu_sc as plsc

D_MODEL = 64
SCALE = math.sqrt(D_MODEL)
NC, NS = 2, 16
NW = NC * NS
LANES = 16
C = 128
NBUF = 2

B0, B1 = 4096, 200
TOTAL = B0 * B1
CH_PER_W = TOTAL // C // NW


@jax.jit
def _emb_lookup(x_flat, table2):
    mesh = plsc.VectorSubcoreMesh(core_axis_name="c", subcore_axis_name="s")

    @functools.partial(
        pl.kernel,
        out_type=jax.ShapeDtypeStruct((TOTAL, D_MODEL), jnp.float32),
        mesh=mesh,
        scratch_types=(
            [pltpu.VMEM((C,), jnp.int32) for _ in range(NBUF)]
            + [pltpu.VMEM((C,), jnp.int32) for _ in range(NBUF)]
            + [pltpu.VMEM((C, 2 * D_MODEL), jnp.float32) for _ in range(NBUF)]
            + [pltpu.VMEM((C, D_MODEL), jnp.float32) for _ in range(NBUF)]
            + [pltpu.SemaphoreType.DMA for _ in range(2 * NBUF)]
        ),
        compiler_params=pltpu.CompilerParams(
            use_tc_tiling_on_sc=True, needs_layout_passes=False
        ),
    )
    def body(x_hbm, table_hbm, out_hbm, *scratch):
        iv = scratch[:NBUF]
        idx2 = scratch[NBUF:2 * NBUF]
        rows2 = scratch[2 * NBUF:3 * NBUF]
        wbuf = scratch[3 * NBUF:4 * NBUF]
        gsem = scratch[4 * NBUF:5 * NBUF]
        osem = scratch[5 * NBUF:]

        wid = lax.axis_index("s") * NC + lax.axis_index("c")
        chunk0 = wid * CH_PER_W

        def start_gather(b, gc):
            start = pl.multiple_of(gc * C, C)
            pltpu.sync_copy(x_hbm.at[pl.ds(start, C)], iv[b])

            def shift(k, carry):
                sl = pl.ds(k * LANES, LANES)
                idx2[b][sl] = lax.shift_right_logical(iv[b][sl], 1)
                return carry

            lax.fori_loop(0, C // LANES, shift, 0)
            pltpu.async_copy(table_hbm.at[idx2[b]], rows2[b], gsem[b])

        def wait_gather(b):
            pltpu.make_async_copy(table_hbm.at[idx2[b]], rows2[b], gsem[b]).wait()

        def select_scale(b):
            def gbody(g, carry):
                base = g * LANES
                hv = (iv[b][pl.ds(base, LANES)] & 1) * D_MODEL
                for kk in range(LANES):
                    k = base + kk
                    h = hv[kk]
                    for j in range(D_MODEL // LANES):
                        v = rows2[b][k, pl.ds(h + j * LANES, LANES)]
                        wbuf[b][k, pl.ds(j * LANES, LANES)] = v * SCALE
                return carry

            lax.fori_loop(0, C // LANES, gbody, 0)

        def start_write(b, gc):
            start = pl.multiple_of(gc * C, C)
            pltpu.async_copy(wbuf[b], out_hbm.at[pl.ds(start, C)], osem[b])

        def wait_write(b, gc):
            start = pl.multiple_of(gc * C, C)
            pltpu.make_async_copy(
                wbuf[b], out_hbm.at[pl.ds(start, C)], osem[b]
            ).wait()

        def process(b, gc, do_wait_write):
            wait_gather(b)
            if do_wait_write:
                wait_write(b, gc - NBUF)
            select_scale(b)
            start_write(b, gc)

        for b in range(NBUF):
            start_gather(b, chunk0 + b)
        for b in range(NBUF):
            process(b, chunk0 + b, do_wait_write=False)
            start_gather(b, chunk0 + b + NBUF)

        def main(i, carry):
            for b in range(NBUF):
                gc = chunk0 + i * NBUF + b
                process(b, gc, do_wait_write=True)
                start_gather(b, gc + NBUF)
            return carry

        lax.fori_loop(1, CH_PER_W // NBUF - 1, main, 0)

        for b in range(NBUF):
            gc = chunk0 + CH_PER_W - NBUF + b
            process(b, gc, do_wait_write=True)
        for b in range(NBUF):
            wait_write(b, chunk0 + CH_PER_W - NBUF + b)

    return body(x_flat, table2)


def kernel(x, table):
    x_flat = x.reshape(TOTAL)
    table2 = table.reshape(-1, 2 * D_MODEL)
    out = _emb_lookup(x_flat, table2)
    return out.reshape(B0, B1, D_MODEL)
